# Initial kernel scaffold; baseline (speedup 1.0000x reference)
import jax
import jax.numpy as jnp
from jax.experimental import pallas as pl


def kernel(x, W_gate, W1, b1, W2, b2):
    raise NotImplementedError("write your pallas kernel here")


# speedup vs baseline: 3.3840x; 3.3840x over previous
import functools

import jax
import jax.numpy as jnp
import numpy as np
from jax.experimental import pallas as pl
from jax.experimental.pallas import tpu as pltpu

H = 1024
DFF = 4096
E = 8
NB = 2
B = 32
S = 32

SEQ_PER_BLK = 8
NUM_BLOCKS = E + (NB * B - E) // SEQ_PER_BLK
NUM_SLOTS = NUM_BLOCKS * SEQ_PER_BLK
TRASH_ROW = NB * B


def _ffn_body(e_of_blk, seq_ids, pos_ids, x_ref, w1_ref, b1_ref, w2_ref,
              b2_ref, wgt_ref, out_ref):
    b = pl.program_id(0)
    xs = [x_ref[seq_ids[SEQ_PER_BLK * b + i]] for i in range(SEQ_PER_BLK)]
    xg = jnp.concatenate(xs, axis=0)
    h = jnp.dot(xg, w1_ref[0], preferred_element_type=jnp.float32)
    h = h + b1_ref[0]
    h = 0.5 * h * (1.0 + jax.lax.erf(h * (1.0 / np.sqrt(2.0).astype(np.float32))))
    c = jnp.dot(h.astype(jnp.bfloat16), w2_ref[0],
                preferred_element_type=jnp.float32)
    b2v = b2_ref[0]
    for i in range(SEQ_PER_BLK):
        pos = pos_ids[SEQ_PER_BLK * b + i]
        w = wgt_ref[0, 0, i]
        val = w * (c[S * i:S * (i + 1), :] + b2v)
        out_ref[pl.ds(pos, 1)] = val[None]


def _run_ffn(x, W1, b1, W2, b2, e_of_blk, seq_ids, pos_ids, w_slot):
    x_bf = x.astype(jnp.bfloat16)
    w1_bf = W1.astype(jnp.bfloat16)
    w2_bf = W2.astype(jnp.bfloat16)
    b1_3d = b1.reshape(E, 1, DFF)
    b2_3d = b2.reshape(E, 1, H)
    w_3d = w_slot.reshape(NUM_BLOCKS, 1, SEQ_PER_BLK)

    grid_spec = pltpu.PrefetchScalarGridSpec(
        num_scalar_prefetch=3,
        grid=(NUM_BLOCKS,),
        in_specs=[
            pl.BlockSpec((B, S, H), lambda b, eb, si, pi: (0, 0, 0)),
            pl.BlockSpec((1, H, DFF), lambda b, eb, si, pi: (eb[b], 0, 0)),
            pl.BlockSpec((1, 1, DFF), lambda b, eb, si, pi: (eb[b], 0, 0)),
            pl.BlockSpec((1, DFF, H), lambda b, eb, si, pi: (eb[b], 0, 0)),
            pl.BlockSpec((1, 1, H), lambda b, eb, si, pi: (eb[b], 0, 0)),
            pl.BlockSpec((1, 1, SEQ_PER_BLK), lambda b, eb, si, pi: (b, 0, 0)),
        ],
        out_specs=pl.BlockSpec((NB * B + 1, S, H), lambda b, eb, si, pi: (0, 0, 0)),
    )
    out = pl.pallas_call(
        _ffn_body,
        grid_spec=grid_spec,
        out_shape=jax.ShapeDtypeStruct((NB * B + 1, S, H), jnp.float32),
    )(e_of_blk, seq_ids, pos_ids, x_bf, w1_bf, b1_3d, w2_bf, b2_3d, w_3d)
    return out[:NB * B]


def kernel(x, W_gate, W1, b1, W2, b2):
    x_avg = jnp.sum(x, axis=1) / jnp.float32(x.shape[1])
    logits_gate = x_avg @ W_gate.T
    prob_gate = jax.nn.softmax(logits_gate, axis=-1)

    importance = jnp.sum(prob_gate, axis=0)
    importance_loss = (jnp.std(importance, ddof=1) / jnp.mean(importance)) ** 2

    current_scores = jnp.exp(jnp.log(prob_gate))
    topk_values, gate = jax.lax.top_k(current_scores, NB)
    beam_scores = topk_values.reshape(NB * B)
    expert_route = gate.reshape(NB * B)[:, None]

    flat_e = gate.reshape(NB * B).astype(jnp.int32)
    order = jnp.argsort(flat_e, stable=True).astype(jnp.int32)
    e_sorted = flat_e[order]
    counts = jnp.bincount(flat_e, length=E)
    blocks_e = (counts + SEQ_PER_BLK - 1) // SEQ_PER_BLK
    blk_base = jnp.concatenate([jnp.zeros((1,), counts.dtype),
                                jnp.cumsum(blocks_e)[:-1]])
    cnt_base = jnp.concatenate([jnp.zeros((1,), counts.dtype),
                                jnp.cumsum(counts)[:-1]])
    r = jnp.arange(NB * B)
    slot = (SEQ_PER_BLK * blk_base[e_sorted] + (r - cnt_base[e_sorted])
            ).astype(jnp.int32)
    seq_ids = jnp.zeros((NUM_SLOTS,), jnp.int32).at[slot].set(order // NB)
    pos_ids = jnp.full((NUM_SLOTS,), TRASH_ROW, jnp.int32).at[slot].set(order)
    w_slot = jnp.zeros((NUM_SLOTS,), jnp.float32).at[slot].set(
        topk_values.reshape(NB * B)[order])
    cum_blocks = jnp.cumsum(blocks_e)
    e_of_blk = jnp.minimum(
        jnp.searchsorted(cum_blocks, jnp.arange(NUM_BLOCKS), side='right'),
        E - 1).astype(jnp.int32)

    output = _run_ffn(x, W1, b1, W2, b2, e_of_blk, seq_ids, pos_ids, w_slot)
    return output, beam_scores, expert_route, importance_loss


# speedup vs baseline: 4.4501x; 1.3151x over previous
import functools

import jax
import jax.numpy as jnp
import numpy as np
from jax.experimental import pallas as pl
from jax.experimental.pallas import tpu as pltpu

H = 1024
DFF = 4096
E = 8
NB = 2
B = 32
S = 32

SEQ_PER_BLK = 8
NUM_BLOCKS = E + (NB * B - E) // SEQ_PER_BLK
NUM_SLOTS = NUM_BLOCKS * SEQ_PER_BLK
TRASH_ROW = NB * B


DFFT = 2048
NT = DFF // DFFT


def _ffn_body(e_of_blk, seq_ids, pos_ids, x_ref, w1_ref, b1_ref, w2_ref,
              b2_ref, wgt_ref, out_ref):
    t = pl.program_id(0)
    b = pl.program_id(1)
    xs = [x_ref[seq_ids[SEQ_PER_BLK * b + i]] for i in range(SEQ_PER_BLK)]
    xg = jnp.concatenate(xs, axis=0).astype(jnp.bfloat16)
    h = jnp.dot(xg, w1_ref[0].astype(jnp.bfloat16),
                preferred_element_type=jnp.float32)
    h = h + b1_ref[0]
    h = 0.5 * h * (1.0 + jax.lax.erf(h * (1.0 / np.sqrt(2.0).astype(np.float32))))
    c = jnp.dot(h.astype(jnp.bfloat16), w2_ref[0].astype(jnp.bfloat16),
                preferred_element_type=jnp.float32)
    b2v = b2_ref[0]
    for i in range(SEQ_PER_BLK):
        pos = pos_ids[SEQ_PER_BLK * b + i]
        w = wgt_ref[0, 0, i]
        val = w * (c[S * i:S * (i + 1), :] + b2v)

        @pl.when(t == 0)
        def _init():
            out_ref[pl.ds(pos, 1)] = val[None]

        @pl.when(t != 0)
        def _acc():
            out_ref[pl.ds(pos, 1)] += (w * c[S * i:S * (i + 1), :])[None]


def _run_ffn(x, W1, b1, W2, b2, e_of_blk, seq_ids, pos_ids, w_slot):
    b1_3d = b1.reshape(E, 1, DFF)
    b2_3d = b2.reshape(E, 1, H)
    w_3d = w_slot.reshape(NUM_BLOCKS, 1, SEQ_PER_BLK)

    grid_spec = pltpu.PrefetchScalarGridSpec(
        num_scalar_prefetch=3,
        grid=(NT, NUM_BLOCKS),
        in_specs=[
            pl.BlockSpec((B, S, H), lambda t, b, eb, si, pi: (0, 0, 0)),
            pl.BlockSpec((1, H, DFFT), lambda t, b, eb, si, pi: (eb[b], 0, t)),
            pl.BlockSpec((1, 1, DFFT), lambda t, b, eb, si, pi: (eb[b], 0, t)),
            pl.BlockSpec((1, DFFT, H), lambda t, b, eb, si, pi: (eb[b], t, 0)),
            pl.BlockSpec((1, 1, H), lambda t, b, eb, si, pi: (eb[b], 0, 0)),
            pl.BlockSpec((1, 1, SEQ_PER_BLK), lambda t, b, eb, si, pi: (b, 0, 0)),
        ],
        out_specs=pl.BlockSpec((NB * B + 1, S, H),
                               lambda t, b, eb, si, pi: (0, 0, 0)),
    )
    out = pl.pallas_call(
        _ffn_body,
        grid_spec=grid_spec,
        out_shape=jax.ShapeDtypeStruct((NB * B + 1, S, H), jnp.float32),
    )(e_of_blk, seq_ids, pos_ids, x, W1, b1_3d, W2, b2_3d, w_3d)
    return out[:NB * B]


def kernel(x, W_gate, W1, b1, W2, b2):
    x_avg = jnp.sum(x, axis=1) / jnp.float32(x.shape[1])
    logits_gate = x_avg @ W_gate.T
    prob_gate = jax.nn.softmax(logits_gate, axis=-1)

    importance = jnp.sum(prob_gate, axis=0)
    importance_loss = (jnp.std(importance, ddof=1) / jnp.mean(importance)) ** 2

    current_scores = jnp.exp(jnp.log(prob_gate))
    topk_values, gate = jax.lax.top_k(current_scores, NB)
    beam_scores = topk_values.reshape(NB * B)
    expert_route = gate.reshape(NB * B)[:, None]

    flat_e = gate.reshape(NB * B).astype(jnp.int32)
    order = jnp.argsort(flat_e, stable=True).astype(jnp.int32)
    e_sorted = flat_e[order]
    counts = jnp.bincount(flat_e, length=E)
    blocks_e = (counts + SEQ_PER_BLK - 1) // SEQ_PER_BLK
    blk_base = jnp.concatenate([jnp.zeros((1,), counts.dtype),
                                jnp.cumsum(blocks_e)[:-1]])
    cnt_base = jnp.concatenate([jnp.zeros((1,), counts.dtype),
                                jnp.cumsum(counts)[:-1]])
    r = jnp.arange(NB * B)
    slot = (SEQ_PER_BLK * blk_base[e_sorted] + (r - cnt_base[e_sorted])
            ).astype(jnp.int32)
    seq_ids = jnp.zeros((NUM_SLOTS,), jnp.int32).at[slot].set(order // NB)
    pos_ids = jnp.full((NUM_SLOTS,), TRASH_ROW, jnp.int32).at[slot].set(order)
    w_slot = jnp.zeros((NUM_SLOTS,), jnp.float32).at[slot].set(
        topk_values.reshape(NB * B)[order])
    cum_blocks = jnp.cumsum(blocks_e)
    e_of_blk = jnp.minimum(
        jnp.searchsorted(cum_blocks, jnp.arange(NUM_BLOCKS), side='right'),
        E - 1).astype(jnp.int32)

    output = _run_ffn(x, W1, b1, W2, b2, e_of_blk, seq_ids, pos_ids, w_slot)
    return output, beam_scores, expert_route, importance_loss


# speedup vs baseline: 4.9512x; 1.1126x over previous
import functools

import jax
import jax.numpy as jnp
import numpy as np
from jax.experimental import pallas as pl
from jax.experimental.pallas import tpu as pltpu

H = 1024
DFF = 4096
E = 8
NB = 2
B = 32
S = 32

SEQ_PER_BLK = 8
NUM_BLOCKS = E + (NB * B - E) // SEQ_PER_BLK
NUM_SLOTS = NUM_BLOCKS * SEQ_PER_BLK
TRASH_ROW = NB * B


DFFT = 2048
NT = DFF // DFFT


def _ffn_body(nreal, e_of_blk, seq_ids, pos_ids, x_ref, w1_ref, b1_ref,
              w2_ref, b2_ref, wgt_ref, out_ref):
    t = pl.program_id(0)
    b = pl.program_id(1)

    @pl.when(b < nreal[0])
    def _work():
        xs = [x_ref[seq_ids[SEQ_PER_BLK * b + i]] for i in range(SEQ_PER_BLK)]
        xg = jnp.concatenate(xs, axis=0).astype(jnp.bfloat16)
        h = jnp.dot(xg, w1_ref[0].astype(jnp.bfloat16),
                    preferred_element_type=jnp.float32)
        h = h + b1_ref[0]
        h = 0.5 * h * (1.0 + jax.lax.erf(
            h * (1.0 / np.sqrt(2.0).astype(np.float32))))
        c = jnp.dot(h.astype(jnp.bfloat16), w2_ref[0].astype(jnp.bfloat16),
                    preferred_element_type=jnp.float32)
        b2v = b2_ref[0]
        for i in range(SEQ_PER_BLK):
            pos = pos_ids[SEQ_PER_BLK * b + i]
            w = wgt_ref[0, 0, i]
            val = w * (c[S * i:S * (i + 1), :] + b2v)

            @pl.when(t == 0)
            def _init():
                out_ref[pl.ds(pos, 1)] = val[None]

            @pl.when(t != 0)
            def _acc():
                out_ref[pl.ds(pos, 1)] += (w * c[S * i:S * (i + 1), :])[None]


def _run_ffn(x, W1, b1, W2, b2, nreal, e_of_blk, seq_ids, pos_ids, w_slot):
    b1_3d = b1.reshape(E, 1, DFF)
    b2_3d = b2.reshape(E, 1, H)
    w_3d = w_slot.reshape(NUM_BLOCKS, 1, SEQ_PER_BLK)

    grid_spec = pltpu.PrefetchScalarGridSpec(
        num_scalar_prefetch=4,
        grid=(NT, NUM_BLOCKS),
        in_specs=[
            pl.BlockSpec((B, S, H), lambda t, b, nr, eb, si, pi: (0, 0, 0)),
            pl.BlockSpec((1, H, DFFT), lambda t, b, nr, eb, si, pi: (eb[b], 0, t)),
            pl.BlockSpec((1, 1, DFFT), lambda t, b, nr, eb, si, pi: (eb[b], 0, t)),
            pl.BlockSpec((1, DFFT, H), lambda t, b, nr, eb, si, pi: (eb[b], t, 0)),
            pl.BlockSpec((1, 1, H), lambda t, b, nr, eb, si, pi: (eb[b], 0, 0)),
            pl.BlockSpec((1, 1, SEQ_PER_BLK),
                         lambda t, b, nr, eb, si, pi: (b, 0, 0)),
        ],
        out_specs=pl.BlockSpec((NB * B + 1, S, H),
                               lambda t, b, nr, eb, si, pi: (0, 0, 0)),
    )
    out = pl.pallas_call(
        _ffn_body,
        grid_spec=grid_spec,
        out_shape=jax.ShapeDtypeStruct((NB * B + 1, S, H), jnp.float32),
    )(nreal, e_of_blk, seq_ids, pos_ids, x, W1, b1_3d, W2, b2_3d, w_3d)
    return out[:NB * B]


def kernel(x, W_gate, W1, b1, W2, b2):
    x_avg = jnp.sum(x, axis=1) / jnp.float32(x.shape[1])
    logits_gate = x_avg @ W_gate.T
    prob_gate = jax.nn.softmax(logits_gate, axis=-1)

    importance = jnp.sum(prob_gate, axis=0)
    importance_loss = (jnp.std(importance, ddof=1) / jnp.mean(importance)) ** 2

    current_scores = jnp.exp(jnp.log(prob_gate))
    topk_values, gate = jax.lax.top_k(current_scores, NB)
    beam_scores = topk_values.reshape(NB * B)
    expert_route = gate.reshape(NB * B)[:, None]

    flat_e = gate.reshape(NB * B).astype(jnp.int32)
    order = jnp.argsort(flat_e, stable=True).astype(jnp.int32)
    e_sorted = flat_e[order]
    counts = jnp.bincount(flat_e, length=E)
    blocks_e = (counts + SEQ_PER_BLK - 1) // SEQ_PER_BLK
    blk_base = jnp.concatenate([jnp.zeros((1,), counts.dtype),
                                jnp.cumsum(blocks_e)[:-1]])
    cnt_base = jnp.concatenate([jnp.zeros((1,), counts.dtype),
                                jnp.cumsum(counts)[:-1]])
    r = jnp.arange(NB * B)
    slot = (SEQ_PER_BLK * blk_base[e_sorted] + (r - cnt_base[e_sorted])
            ).astype(jnp.int32)
    seq_ids = jnp.zeros((NUM_SLOTS,), jnp.int32).at[slot].set(order // NB)
    pos_ids = jnp.full((NUM_SLOTS,), TRASH_ROW, jnp.int32).at[slot].set(order)
    w_slot = jnp.zeros((NUM_SLOTS,), jnp.float32).at[slot].set(
        topk_values.reshape(NB * B)[order])
    cum_blocks = jnp.cumsum(blocks_e)
    e_raw = jnp.searchsorted(cum_blocks, jnp.arange(NUM_BLOCKS), side='right')
    e_of_blk = jnp.where(jnp.arange(NUM_BLOCKS) < cum_blocks[-1],
                         e_raw, e_sorted[-1]).astype(jnp.int32)
    nreal = cum_blocks[-1:].astype(jnp.int32)

    output = _run_ffn(x, W1, b1, W2, b2, nreal, e_of_blk, seq_ids, pos_ids,
                      w_slot)
    return output, beam_scores, expert_route, importance_loss
